# Initial kernel scaffold; baseline (speedup 1.0000x reference)
import jax
import jax.numpy as jnp
from jax.experimental import pallas as pl


def kernel(x, edge_index, W1, b1, W2, b2, fW1, fb1, fW2, fb2, fW3, fb3):
    raise NotImplementedError("write your pallas kernel here")


# speedup vs baseline: 13.8547x; 13.8547x over previous
import functools

import jax
import jax.numpy as jnp
from jax import lax
from jax.experimental import pallas as pl
from jax.experimental.pallas import tpu as pltpu
from jax.experimental.pallas import tpu_sc as plsc

N = 10000
F_IN = 256
H = 64
E = 160000

NC = 2
NS = 16
NW = NC * NS
EPW = E // NW
CH = 125
NCHUNK = EPW // CH
RPT = 624
TAIL = N - NS * RPT
DEGW = 16

_sc_mesh = plsc.VectorSubcoreMesh(
    core_axis_name="c", subcore_axis_name="s", num_cores=NC, num_subcores=NS)
_sc_params = pltpu.CompilerParams(use_tc_tiling_on_sc=False)


@functools.partial(
    pl.kernel,
    mesh=_sc_mesh,
    out_type=jax.ShapeDtypeStruct((NC, N, DEGW), jnp.float32),
    compiler_params=_sc_params,
    scratch_types=[
        pltpu.VMEM((NCHUNK, CH), jnp.int32),
        pltpu.VMEM((CH, DEGW), jnp.float32),
        pltpu.VMEM_SHARED((N, DEGW), jnp.float32),
    ],
)
def _sc_degree(dst_hbm, ones_hbm, zeros_hbm, out_hbm, dst_v, ones_v, deg_sh):
    cid = lax.axis_index("c")
    sid = lax.axis_index("s")
    wid = sid * NC + cid
    pltpu.sync_copy(dst_hbm.at[wid], dst_v)
    pltpu.sync_copy(ones_hbm, ones_v)
    pltpu.sync_copy(zeros_hbm, deg_sh.at[pl.ds(sid * RPT, RPT)])

    @pl.when(sid == 0)
    def _():
        pltpu.sync_copy(zeros_hbm.at[pl.ds(0, TAIL)],
                        deg_sh.at[pl.ds(NS * RPT, TAIL)])

    plsc.subcore_barrier()

    def body(j, carry):
        pltpu.sync_copy(ones_v, deg_sh.at[dst_v.at[j]], add=True)
        return carry

    lax.fori_loop(0, NCHUNK, body, 0)
    plsc.subcore_barrier()
    pltpu.sync_copy(deg_sh.at[pl.ds(sid * RPT, RPT)],
                    out_hbm.at[cid, pl.ds(sid * RPT, RPT)])

    @pl.when(sid == 0)
    def _():
        pltpu.sync_copy(deg_sh.at[pl.ds(NS * RPT, TAIL)],
                        out_hbm.at[cid, pl.ds(NS * RPT, TAIL)])


@functools.partial(
    pl.kernel,
    mesh=_sc_mesh,
    out_type=jax.ShapeDtypeStruct((NC, N, H), jnp.float32),
    compiler_params=_sc_params,
    scratch_types=[
        pltpu.VMEM((NCHUNK, CH), jnp.int32),
        pltpu.VMEM((NCHUNK, CH), jnp.int32),
        pltpu.VMEM((CH, H), jnp.float32),
        pltpu.VMEM_SHARED((N, H), jnp.float32),
        pltpu.SemaphoreType.DMA,
    ],
)
def _sc_aggregate(src_hbm, dst_hbm, y_hbm, zeros_hbm, out_hbm,
                  src_v, dst_v, rows_v, agg_sh, sem):
    cid = lax.axis_index("c")
    sid = lax.axis_index("s")
    wid = sid * NC + cid
    pltpu.sync_copy(src_hbm.at[wid], src_v)
    pltpu.sync_copy(dst_hbm.at[wid], dst_v)
    pltpu.sync_copy(zeros_hbm, agg_sh.at[pl.ds(sid * RPT, RPT)])

    @pl.when(sid == 0)
    def _():
        pltpu.sync_copy(zeros_hbm.at[pl.ds(0, TAIL)],
                        agg_sh.at[pl.ds(NS * RPT, TAIL)])

    plsc.subcore_barrier()

    def body(j, carry):
        pltpu.async_copy(y_hbm.at[src_v.at[j]], rows_v, sem).wait()
        pltpu.sync_copy(rows_v, agg_sh.at[dst_v.at[j]], add=True)
        return carry

    lax.fori_loop(0, NCHUNK, body, 0)
    plsc.subcore_barrier()
    pltpu.sync_copy(agg_sh.at[pl.ds(sid * RPT, RPT)],
                    out_hbm.at[cid, pl.ds(sid * RPT, RPT)])

    @pl.when(sid == 0)
    def _():
        pltpu.sync_copy(agg_sh.at[pl.ds(NS * RPT, TAIL)],
                        out_hbm.at[cid, pl.ds(NS * RPT, TAIL)])


BN = 1000


def _dinv_block(deg_ref):
    deg = deg_ref[0, :, 0:1] + deg_ref[1, :, 0:1] + 1.0
    return lax.rsqrt(deg)


def _tc_scale1_body(deg_ref, x_ref, w_ref, y_ref):
    dinv = _dinv_block(deg_ref)
    xw = jnp.dot(x_ref[...], w_ref[...], preferred_element_type=jnp.float32)
    y_ref[...] = xw * dinv


def _tc_layer2_body(deg_ref, agg_ref, y1_ref, b_ref, w_ref, y2_ref):
    dinv = _dinv_block(deg_ref)
    h = agg_ref[0] + agg_ref[1] + y1_ref[...]
    h = jnp.maximum(h * dinv + b_ref[...], 0.0)
    y2_ref[...] = jnp.dot(h, w_ref[...],
                          preferred_element_type=jnp.float32) * dinv


def _tc_h2_body(deg_ref, agg_ref, y2_ref, b_ref, h_ref):
    dinv = _dinv_block(deg_ref)
    h = agg_ref[0] + agg_ref[1] + y2_ref[...]
    h_ref[...] = jnp.maximum(h * dinv + b_ref[...], 0.0)


BK = 16000
NKB = (N * H) // BK


def _tc_head_body(h_ref, w1_ref, b1_ref, w2_ref, b2_ref, w3_ref, b3_ref,
                  out_ref, acc_ref):
    k = pl.program_id(0)

    @pl.when(k == 0)
    def _():
        acc_ref[...] = jnp.zeros_like(acc_ref)

    acc_ref[...] += jnp.dot(h_ref[...], w1_ref[...],
                            preferred_element_type=jnp.float32)

    @pl.when(k == NKB - 1)
    def _():
        z1 = jnp.maximum(acc_ref[...] + b1_ref[...], 0.0)
        z2 = jnp.maximum(
            jnp.dot(z1, w2_ref[...], preferred_element_type=jnp.float32)
            + b2_ref[...], 0.0)
        out_ref[...] = jnp.dot(
            z2, w3_ref[...], preferred_element_type=jnp.float32,
            precision=lax.Precision.HIGHEST) + b3_ref[...]


def _run(x, edge_index, W1, b1, W2, b2, fW1, fb1, fW2, fb2, fW3, fb3):
    f32 = jnp.float32
    src = edge_index[0].reshape(NW, NCHUNK, CH)
    dst = edge_index[1].reshape(NW, NCHUNK, CH)
    zeros_h = jnp.zeros((RPT, H), f32)
    zeros_d = jnp.zeros((RPT, DEGW), f32)
    ones_d = jnp.ones((CH, DEGW), f32)

    deg_parts = _sc_degree(dst, ones_d, zeros_d)

    grid_n = N // BN
    deg_spec = pl.BlockSpec((NC, BN, DEGW), lambda i: (0, i, 0))
    row_h = pl.BlockSpec((BN, H), lambda i: (i, 0))
    agg_spec = pl.BlockSpec((NC, BN, H), lambda i: (0, i, 0))
    full = lambda *shape: pl.BlockSpec(shape, lambda i: (0,) * len(shape))

    y1 = pl.pallas_call(
        _tc_scale1_body,
        grid=(grid_n,),
        in_specs=[deg_spec,
                  pl.BlockSpec((BN, F_IN), lambda i: (i, 0)),
                  full(F_IN, H)],
        out_specs=row_h,
        out_shape=jax.ShapeDtypeStruct((N, H), f32),
    )(deg_parts, x, W1)

    agg1 = _sc_aggregate(src, dst, y1, zeros_h)

    y2 = pl.pallas_call(
        _tc_layer2_body,
        grid=(grid_n,),
        in_specs=[deg_spec, agg_spec, row_h, full(1, H), full(H, H)],
        out_specs=row_h,
        out_shape=jax.ShapeDtypeStruct((N, H), f32),
    )(deg_parts, agg1, y1, b1.reshape(1, H), W2)

    agg2 = _sc_aggregate(src, dst, y2, zeros_h)

    h2 = pl.pallas_call(
        _tc_h2_body,
        grid=(grid_n,),
        in_specs=[deg_spec, agg_spec, row_h, full(1, H)],
        out_specs=row_h,
        out_shape=jax.ShapeDtypeStruct((N, H), f32),
    )(deg_parts, agg2, y2, b2.reshape(1, H))

    out = pl.pallas_call(
        _tc_head_body,
        grid=(NKB,),
        in_specs=[pl.BlockSpec((1, BK), lambda k: (0, k)),
                  pl.BlockSpec((BK, 128), lambda k: (k, 0)),
                  full(1, 128), full(128, 64), full(1, 64),
                  full(64, 1), full(1, 1)],
        out_specs=pl.BlockSpec((1, 1), lambda k: (0, 0)),
        out_shape=jax.ShapeDtypeStruct((1, 1), f32),
        scratch_shapes=[pltpu.VMEM((1, 128), f32)],
        compiler_params=pltpu.CompilerParams(
            dimension_semantics=("arbitrary",)),
    )(h2.reshape(1, N * H), fW1, fb1.reshape(1, 128), fW2,
      fb2.reshape(1, 64), fW3, fb3.reshape(1, 1))

    return out, {"deg": deg_parts, "y1": y1, "agg1": agg1, "y2": y2,
                 "agg2": agg2, "h2": h2}


def _head(h2f, fW1, fb1, fW2, fb2, fW3, fb3):
    full = lambda *shape: pl.BlockSpec(shape, lambda k: (0,) * len(shape))
    return pl.pallas_call(
        _tc_head_body,
        grid=(NKB,),
        in_specs=[pl.BlockSpec((1, BK), lambda k: (0, k)),
                  pl.BlockSpec((BK, 128), lambda k: (k, 0)),
                  full(1, 128), full(128, 64), full(1, 64),
                  full(64, 1), full(1, 1)],
        out_specs=pl.BlockSpec((1, 1), lambda k: (0, 0)),
        out_shape=jax.ShapeDtypeStruct((1, 1), jnp.float32),
        scratch_shapes=[pltpu.VMEM((1, 128), jnp.float32)],
        compiler_params=pltpu.CompilerParams(
            dimension_semantics=("arbitrary",)),
    )(h2f, fW1, fb1.reshape(1, 128), fW2,
      fb2.reshape(1, 64), fW3, fb3.reshape(1, 1))


def kernel(x, edge_index, W1, b1, W2, b2, fW1, fb1, fW2, fb2, fW3, fb3):
    return _run(x, edge_index, W1, b1, W2, b2,
                fW1, fb1, fW2, fb2, fW3, fb3)[0]
